# baseline (speedup 1.0000x reference)
import jax
import jax.numpy as jnp
from jax import lax
from jax.experimental import pallas as pl
from jax.experimental.pallas import tpu as pltpu
from jax.experimental.pallas import tpu_sc as plsc

_B = 4
_N_FINE = 163842
_N_COARSE = 40962
_C = 128
_K = 7

_NW = 32
_T = 48
_NT = 107
_TOTCH = _NW * _NT
_RTOT = _B * _N_COARSE
_NREAL = _RTOT // _T + 1
_LAST_START = _RTOT - _T
_IPC = _T * _K
_IDX_MINOR = 112
_NSEG = _IPC // _IDX_MINOR


def _pool_body(x_hbm, idx_hbm, out_hbm,
               idx_v0, idx_v1, rows_v0, rows_v1, out_v0, out_v1,
               isem0, isem1, gsem0, gsem1, osem0, osem1):
    wid = lax.axis_index("s") * 2 + lax.axis_index("c")
    idx_v = (idx_v0, idx_v1)
    rows_v = (rows_v0, rows_v1)
    out_v = (out_v0, out_v1)
    isem = (isem0, isem1)
    gsem = (gsem0, gsem1)
    osem = (osem0, osem1)

    def chunk_id(t):
        return t * _NW + wid

    def fire_gathers(b, cw):
        for k in range(_NSEG):
            pltpu.async_copy(
                x_hbm.at[idx_v[b].at[k]],
                rows_v[b].at[pl.ds(k * _IDX_MINOR, _IDX_MINOR)],
                gsem[b],
            )

    def wait_gathers(b):
        with jax.named_scope("gwait"):
            pltpu.make_async_copy(
                x_hbm.at[pl.ds(0, _IPC)], rows_v[b], gsem[b]
            ).wait()

    def out_start(cw):
        return jnp.minimum(cw * _T, _LAST_START)

    def compute_and_write(b, cw):
        rows = rows_v[b]
        outb = out_v[b]

        with jax.named_scope("compute"):
            @plsc.parallel_loop(0, _T, step=1, unroll=2)
            def row_body(r):
                base = r * _K
                for j in range(_C // 16):
                    sl = pl.ds(j * 16, 16)
                    a0 = rows[base, sl] + rows[base + 1, sl]
                    a1 = rows[base + 2, sl] + rows[base + 3, sl]
                    a2 = rows[base + 4, sl] + rows[base + 5, sl]
                    acc = (a0 + a1) + (a2 + rows[base + 6, sl])
                    outb[r, sl] = acc * (1.0 / _K)

        @pl.when(cw < _NREAL)
        def _():
            pltpu.async_copy(
                outb, out_hbm.at[pl.ds(out_start(cw), _T)], osem[b])

    def wait_out(b, cw):
        @pl.when(cw < _NREAL)
        def _():
            pltpu.make_async_copy(
                out_v[b], out_hbm.at[pl.ds(0, _T)], osem[b]).wait()

    pltpu.sync_copy(idx_hbm.at[chunk_id(0)], idx_v0)
    fire_gathers(0, chunk_id(0))
    pltpu.async_copy(idx_hbm.at[chunk_id(1)], idx_v1, isem1)

    def step(tp, _):
        for b in (0, 1):
            tt = 2 * tp + b
            cw = chunk_id(tt)
            nb = 1 - b

            pltpu.make_async_copy(
                idx_hbm.at[chunk_id(tt + 1)], idx_v[nb], isem[nb]
            ).wait()
            fire_gathers(nb, chunk_id(tt + 1))

            wait_gathers(b)

            @pl.when(tt + 2 < _NT)
            def _():
                pltpu.async_copy(
                    idx_hbm.at[chunk_id(tt + 2)], idx_v[b], isem[b])

            @pl.when(tt >= 2)
            def _():
                wait_out(b, chunk_id(tt - 2))

            compute_and_write(b, cw)
        return 0

    lax.fori_loop(0, _NT // 2, step, 0)

    wait_gathers(0)
    wait_out(0, chunk_id(_NT - 3))
    compute_and_write(0, chunk_id(_NT - 1))

    wait_out(1, chunk_id(_NT - 2))
    wait_out(0, chunk_id(_NT - 1))


@jax.jit
def kernel(x, pool_idx):
    x_flat = x.reshape(_B * _N_FINE, _C)
    offs = (jnp.arange(_B, dtype=jnp.int32) * _N_FINE)[:, None, None]
    flat_idx = (pool_idx[None, :, :] + offs).reshape(-1)
    main = flat_idx[: (_NREAL - 1) * _IPC].reshape(_NREAL - 1, _IPC)
    tail = flat_idx[_LAST_START * _K:]
    tail = jnp.broadcast_to(tail, (_TOTCH - _NREAL + 1, _IPC))
    idx3 = jnp.concatenate([main, tail]).reshape(_TOTCH, _NSEG, _IDX_MINOR)

    mesh = plsc.VectorSubcoreMesh(core_axis_name="c", subcore_axis_name="s")
    out = pl.kernel(
        _pool_body,
        out_type=jax.ShapeDtypeStruct((_RTOT, _C), jnp.float32),
        mesh=mesh,
        scratch_types=[
            pltpu.VMEM((_NSEG, _IDX_MINOR), jnp.int32),
            pltpu.VMEM((_NSEG, _IDX_MINOR), jnp.int32),
            pltpu.VMEM((_IPC, _C), jnp.float32),
            pltpu.VMEM((_IPC, _C), jnp.float32),
            pltpu.VMEM((_T, _C), jnp.float32),
            pltpu.VMEM((_T, _C), jnp.float32),
            pltpu.SemaphoreType.DMA,
            pltpu.SemaphoreType.DMA,
            pltpu.SemaphoreType.DMA,
            pltpu.SemaphoreType.DMA,
            pltpu.SemaphoreType.DMA,
            pltpu.SemaphoreType.DMA,
        ],
    )(x_flat, idx3)
    return out.reshape(_B, _N_COARSE, _C)


# speedup vs baseline: 2.6368x; 2.6368x over previous
import jax
import jax.numpy as jnp
from jax import lax
from jax.experimental import pallas as pl
from jax.experimental.pallas import tpu as pltpu
from jax.experimental.pallas import tpu_sc as plsc

_B = 4
_N_FINE = 163842
_N_COARSE = 40962
_C = 128
_K = 7

_NW = 32
_T = 15
_NT = 87
_TOTCH = _NW * _NT
_NREAL = (_N_COARSE + _T - 1) // _T
_LAST_START = _N_COARSE - _T
_IPC = _T * _K
_IROW = 112


def _pool_body(x_hbm, idx_hbm, out_hbm,
               idx_v0, idx_v1, rows_v0, rows_v1, out_v0, out_v1,
               isem0, isem1, gsem0, gsem1, osem0, osem1):
    wid = lax.axis_index("s") * 2 + lax.axis_index("c")
    idx_v = (idx_v0, idx_v1)
    rows_v = (rows_v0, rows_v1)
    out_v = (out_v0, out_v1)
    isem = (isem0, isem1)
    gsem = (gsem0, gsem1)
    osem = (osem0, osem1)

    def chunk_id(t):
        return t * _NW + wid

    def fire_gather(b):
        pltpu.async_copy(
            x_hbm.at[idx_v[b].at[pl.ds(0, _IPC)]], rows_v[b], gsem[b])

    def wait_gather(b):
        pltpu.make_async_copy(
            x_hbm.at[pl.ds(0, _IPC)], rows_v[b], gsem[b]).wait()

    def out_start(cw):
        return jnp.minimum(cw * _T, _LAST_START)

    def compute_and_write(b, cw):
        rows = rows_v[b]
        outb = out_v[b]

        @plsc.parallel_loop(0, _T, step=1, unroll=2)
        def row_body(r):
            base = r * _K
            for bb in range(_B):
                for j in range(_C // 16):
                    sl = pl.ds(j * 16, 16)
                    a0 = rows[base, bb, sl] + rows[base + 1, bb, sl]
                    a1 = rows[base + 2, bb, sl] + rows[base + 3, bb, sl]
                    a2 = rows[base + 4, bb, sl] + rows[base + 5, bb, sl]
                    acc = (a0 + a1) + (a2 + rows[base + 6, bb, sl])
                    outb[r, bb, sl] = acc * (1.0 / _K)

        @pl.when(cw < _NREAL)
        def _():
            pltpu.async_copy(
                outb, out_hbm.at[pl.ds(out_start(cw), _T)], osem[b])

    def wait_out(b, cw):
        @pl.when(cw < _NREAL)
        def _():
            pltpu.make_async_copy(
                out_v[b], out_hbm.at[pl.ds(0, _T)], osem[b]).wait()

    pltpu.sync_copy(idx_hbm.at[chunk_id(0)], idx_v0)
    fire_gather(0)
    pltpu.async_copy(idx_hbm.at[chunk_id(1)], idx_v1, isem1)

    def step(tp, _):
        for b in (0, 1):
            tt = 2 * tp + b
            cw = chunk_id(tt)
            nb = 1 - b

            pltpu.make_async_copy(
                idx_hbm.at[chunk_id(tt + 1)], idx_v[nb], isem[nb]).wait()
            fire_gather(nb)

            wait_gather(b)

            @pl.when(tt + 2 < _NT)
            def _():
                pltpu.async_copy(
                    idx_hbm.at[chunk_id(tt + 2)], idx_v[b], isem[b])

            @pl.when(tt >= 2)
            def _():
                wait_out(b, chunk_id(tt - 2))

            compute_and_write(b, cw)
        return 0

    lax.fori_loop(0, _NT // 2, step, 0)

    wait_gather(0)
    wait_out(0, chunk_id(_NT - 3))
    compute_and_write(0, chunk_id(_NT - 1))

    wait_out(1, chunk_id(_NT - 2))
    wait_out(0, chunk_id(_NT - 1))


@jax.jit
def kernel(x, pool_idx):
    x_t = x.transpose(1, 0, 2)

    flat = pool_idx.reshape(-1)
    main = flat[: (_NREAL - 1) * _IPC].reshape(_NREAL - 1, _IPC)
    tail = flat[_LAST_START * _K:]
    tail = jnp.broadcast_to(tail, (_TOTCH - _NREAL + 1, _IPC))
    idx3 = jnp.concatenate([main, tail])
    idx3 = jnp.pad(idx3, ((0, 0), (0, _IROW - _IPC)))

    mesh = plsc.VectorSubcoreMesh(core_axis_name="c", subcore_axis_name="s")
    out = pl.kernel(
        _pool_body,
        out_type=jax.ShapeDtypeStruct((_N_COARSE, _B, _C), jnp.float32),
        mesh=mesh,
        scratch_types=[
            pltpu.VMEM((_IROW,), jnp.int32),
            pltpu.VMEM((_IROW,), jnp.int32),
            pltpu.VMEM((_IPC, _B, _C), jnp.float32),
            pltpu.VMEM((_IPC, _B, _C), jnp.float32),
            pltpu.VMEM((_T, _B, _C), jnp.float32),
            pltpu.VMEM((_T, _B, _C), jnp.float32),
            pltpu.SemaphoreType.DMA,
            pltpu.SemaphoreType.DMA,
            pltpu.SemaphoreType.DMA,
            pltpu.SemaphoreType.DMA,
            pltpu.SemaphoreType.DMA,
            pltpu.SemaphoreType.DMA,
        ],
    )(x_t, idx3)
    return out.transpose(1, 0, 2)


# speedup vs baseline: 2.8963x; 1.0984x over previous
import jax
import jax.numpy as jnp
from jax import lax
from jax.experimental import pallas as pl
from jax.experimental.pallas import tpu as pltpu
from jax.experimental.pallas import tpu_sc as plsc

_B = 4
_N_FINE = 163842
_N_COARSE = 40962
_C = 128
_K = 7

_NW = 32
_T = 15
_NT = 87
_TOTCH = _NW * _NT
_NREAL = (_N_COARSE + _T - 1) // _T
_LAST_START = _N_COARSE - _T
_IPC = _T * _K
_IROW = 112


def _pool_body(x_hbm, idx_hbm, out_hbm,
               idx_v0, idx_v1, rows_v0, rows_v1, out_v0, out_v1,
               isem0, isem1, gsem0, gsem1, osem0, osem1):
    wid = lax.axis_index("s") * 2 + lax.axis_index("c")
    idx_v = (idx_v0, idx_v1)
    rows_v = (rows_v0, rows_v1)
    out_v = (out_v0, out_v1)
    isem = (isem0, isem1)
    gsem = (gsem0, gsem1)
    osem = (osem0, osem1)

    def chunk_id(t):
        return t * _NW + wid

    def fire_gather(b):
        pltpu.async_copy(
            x_hbm.at[idx_v[b].at[pl.ds(0, _IPC)]], rows_v[b], gsem[b])

    def wait_gather(b):
        pltpu.make_async_copy(
            x_hbm.at[pl.ds(0, _IPC)], rows_v[b], gsem[b]).wait()

    def out_start(cw):
        return jnp.minimum(cw * _T, _LAST_START)

    def compute_and_write(b, cw):
        rows = rows_v[b]
        outb = out_v[b]

        @plsc.parallel_loop(0, _T, step=1, unroll=2)
        def row_body(r):
            base = r * _K
            for bb in range(_B):
                for j in range(_C // 16):
                    sl = pl.ds(j * 16, 16)
                    a0 = rows[base, bb, sl] + rows[base + 1, bb, sl]
                    a1 = rows[base + 2, bb, sl] + rows[base + 3, bb, sl]
                    a2 = rows[base + 4, bb, sl] + rows[base + 5, bb, sl]
                    acc = (a0 + a1) + (a2 + rows[base + 6, bb, sl])
                    outb[r, bb, sl] = acc * (1.0 / _K)

        @pl.when(cw < _NREAL)
        def _():
            pltpu.async_copy(
                outb, out_hbm.at[pl.ds(out_start(cw), _T)], osem[b])

    def wait_out(b, cw):
        @pl.when(cw < _NREAL)
        def _():
            pltpu.make_async_copy(
                out_v[b], out_hbm.at[pl.ds(0, _T)], osem[b]).wait()

    pltpu.sync_copy(idx_hbm.at[chunk_id(0)], idx_v0)
    fire_gather(0)
    pltpu.async_copy(idx_hbm.at[chunk_id(1)], idx_v1, isem1)

    def step(tp, _):
        for b in (0, 1):
            tt = 2 * tp + b
            cw = chunk_id(tt)
            nb = 1 - b

            pltpu.make_async_copy(
                idx_hbm.at[chunk_id(tt + 1)], idx_v[nb], isem[nb]).wait()
            fire_gather(nb)

            wait_gather(b)

            @pl.when(tt + 2 < _NT)
            def _():
                pltpu.async_copy(
                    idx_hbm.at[chunk_id(tt + 2)], idx_v[b], isem[b])

            @pl.when(tt >= 2)
            def _():
                wait_out(b, chunk_id(tt - 2))

            compute_and_write(b, cw)
        return 0

    lax.fori_loop(0, _NT // 2, step, 0)

    wait_gather(0)
    wait_out(0, chunk_id(_NT - 3))
    compute_and_write(0, chunk_id(_NT - 1))

    wait_out(1, chunk_id(_NT - 2))
    wait_out(0, chunk_id(_NT - 1))


@jax.jit
def kernel(x, pool_idx):
    x_t = x.transpose(1, 0, 2)

    idxt = pool_idx.T
    main = idxt[:, : (_NREAL - 1) * _T].reshape(_K, _NREAL - 1, _T)
    main = main.transpose(1, 2, 0).reshape(_NREAL - 1, _IPC)
    tail = idxt[:, _LAST_START:].transpose(1, 0).reshape(1, _IPC)
    tail = jnp.broadcast_to(tail, (_TOTCH - _NREAL + 1, _IPC))
    idx3 = jnp.concatenate([main, tail])
    idx3 = jnp.pad(idx3, ((0, 0), (0, _IROW - _IPC)))

    mesh = plsc.VectorSubcoreMesh(core_axis_name="c", subcore_axis_name="s")
    out = pl.kernel(
        _pool_body,
        out_type=jax.ShapeDtypeStruct((_N_COARSE, _B, _C), jnp.float32),
        mesh=mesh,
        scratch_types=[
            pltpu.VMEM((_IROW,), jnp.int32),
            pltpu.VMEM((_IROW,), jnp.int32),
            pltpu.VMEM((_IPC, _B, _C), jnp.float32),
            pltpu.VMEM((_IPC, _B, _C), jnp.float32),
            pltpu.VMEM((_T, _B, _C), jnp.float32),
            pltpu.VMEM((_T, _B, _C), jnp.float32),
            pltpu.SemaphoreType.DMA,
            pltpu.SemaphoreType.DMA,
            pltpu.SemaphoreType.DMA,
            pltpu.SemaphoreType.DMA,
            pltpu.SemaphoreType.DMA,
            pltpu.SemaphoreType.DMA,
        ],
    )(x_t, idx3)
    return out.transpose(1, 0, 2)
